# baseline (speedup 1.0000x reference)
import functools

import jax
import jax.numpy as jnp
from jax import lax
from jax.experimental import pallas as pl
from jax.experimental.pallas import tpu as pltpu
from jax.experimental.pallas import tpu_sc as plsc

B = 16384
D = 64
MARGIN = 1.0
NC = 2
NS = 16
NW = NC * NS
BPW = B // NW
NG = BPW // 16


def _tec_body(ph_h, pt_h, pr_h, nh_h, nt_h, nr_h, ent_h, rel_h, out_h,
              ph_i, pt_i, pr_i, nh_i, nt_i, nr_i,
              ra0, ra1, ra2, ra3, ra4, ra5,
              rb0, rb1, rb2, rb3, rb4, rb5,
              ov, sem):
    wid = lax.axis_index("s") * NC + lax.axis_index("c")

    pltpu.sync_copy(ph_h.at[wid], ph_i)
    pltpu.sync_copy(pt_h.at[wid], pt_i)
    pltpu.sync_copy(pr_h.at[wid], pr_i)
    pltpu.sync_copy(nh_h.at[wid], nh_i)
    pltpu.sync_copy(nt_h.at[wid], nt_i)
    pltpu.sync_copy(nr_h.at[wid], nr_i)

    idx_refs = (ph_i, pt_i, pr_i, nh_i, nt_i, nr_i)
    tabs = (ent_h, ent_h, rel_h, ent_h, ent_h, rel_h)
    slots = ((ra0, ra1, ra2, ra3, ra4, ra5), (rb0, rb1, rb2, rb3, rb4, rb5))

    def fire(g, slot):
        r, c0 = g >> 2, (g & 3) * 16
        for j in range(6):
            iv = idx_refs[j][r, pl.ds(c0, 16)]
            for i in range(16):
                pltpu.async_copy(
                    tabs[j].at[iv[i]], slots[slot][j].at[i], sem
                )

    def drain(slot):
        for j in range(6):
            pltpu.make_async_copy(
                tabs[j].at[pl.ds(0, 16)], slots[slot][j], sem
            ).wait()

    def score16(slot):
        bufs = slots[slot]
        total = 0.0
        for i in range(16):
            acc = jnp.zeros((16,), jnp.float32)
            for k in range(4):
                sl = pl.ds(k * 16, 16)
                hp = bufs[0][i, sl]
                tp = bufs[1][i, sl]
                rp = bufs[2][i, sl]
                hn = bufs[3][i, sl]
                tn = bufs[4][i, sl]
                rn = bufs[5][i, sl]
                acc = acc + (jnp.abs(hp + rp - tp) - jnp.abs(hn + rn - tn))
            total = total + jnp.maximum(jnp.sum(acc) + MARGIN, 0.0)
        return total

    fire(0, 0)

    def body(p, loss):
        g = p * 2
        fire(g + 1, 1)
        drain(0)
        loss = loss + score16(0)
        pl.when(g + 2 < NG)(lambda: fire(g + 2, 0))
        drain(1)
        return loss + score16(1)

    loss = lax.fori_loop(0, NG // 2, body, 0.0)

    z16 = jnp.zeros((16,), jnp.float32)
    for r in range(8):
        for kk in range(4):
            ov[r, pl.ds(kk * 16, 16)] = z16
    ov[0, pl.ds(0, 16)] = jnp.full((16,), loss * 0.0625, jnp.float32)
    pltpu.sync_copy(ov, out_h.at[wid])


@functools.partial(jax.jit, static_argnums=())
def _run(ph, pt, pr, nh, nt, nr, ent, rel):
    mesh = plsc.VectorSubcoreMesh(core_axis_name="c", subcore_axis_name="s")
    k = pl.kernel(
        _tec_body,
        mesh=mesh,
        compiler_params=pltpu.CompilerParams(
            needs_layout_passes=False, use_tc_tiling_on_sc=True
        ),
        out_type=jax.ShapeDtypeStruct((NW, 8, 64), jnp.float32),
        scratch_types=(
            [pltpu.VMEM((8, 64), jnp.int32) for _ in range(6)]
            + [pltpu.VMEM((16, 64), jnp.float32) for _ in range(12)]
            + [pltpu.VMEM((8, 64), jnp.float32), pltpu.SemaphoreType.DMA]
        ),
    )
    return k(ph, pt, pr, nh, nt, nr, ent, rel)


BLK = 4096


def _xpose_body(src_ref, dst_ref):
    dst_ref[...] = src_ref[...].T


@jax.jit
def _tc_transpose(xt):
    v = xt.shape[1]
    return pl.pallas_call(
        _xpose_body,
        grid=(pl.cdiv(v, BLK),),
        in_specs=[pl.BlockSpec((D, BLK), lambda i: (0, i))],
        out_specs=pl.BlockSpec((BLK, D), lambda i: (i, 0)),
        out_shape=jax.ShapeDtypeStruct((v, D), jnp.float32),
    )(xt)


def kernel(pos_h, pos_t, pos_r, neg_h, neg_t, neg_r, ent_embeddings, rel_embeddings):
    shp = (NW, 8, 64)
    ph = pos_h.reshape(shp).astype(jnp.int32)
    pt = pos_t.reshape(shp).astype(jnp.int32)
    pr = pos_r.reshape(shp).astype(jnp.int32)
    nh = neg_h.reshape(shp).astype(jnp.int32)
    nt = neg_t.reshape(shp).astype(jnp.int32)
    nr = neg_r.reshape(shp).astype(jnp.int32)
    ent_rm = _tc_transpose(ent_embeddings.T)
    out = _run(ph, pt, pr, nh, nt, nr, ent_rm, rel_embeddings)
    return jnp.sum(out)


# speedup vs baseline: 1.0256x; 1.0256x over previous
import functools

import jax
import jax.numpy as jnp
from jax import lax
from jax.experimental import pallas as pl
from jax.experimental.pallas import tpu as pltpu
from jax.experimental.pallas import tpu_sc as plsc

B = 16384
D = 64
MARGIN = 1.0
NC = 2
NS = 16
NW = NC * NS
BPW = B // NW
NG = BPW // 16
BLK = 4096


def _tec_body(ph_h, pt_h, pr_h, nh_h, nt_h, nr_h, ent_h, rel_h, out_h,
              ph_i, pt_i, pr_i, nh_i, nt_i, nr_i,
              ea0, ea1, ea2, ea3, eb0, eb1, eb2, eb3,
              la0, la1, lb0, lb1,
              ov, sem):
    wid = lax.axis_index("s") * NC + lax.axis_index("c")

    pltpu.sync_copy(ph_h.at[wid], ph_i)
    pltpu.sync_copy(pt_h.at[wid], pt_i)
    pltpu.sync_copy(pr_h.at[wid], pr_i)
    pltpu.sync_copy(nh_h.at[wid], nh_i)
    pltpu.sync_copy(nt_h.at[wid], nt_i)
    pltpu.sync_copy(nr_h.at[wid], nr_i)

    ent_idx = (ph_i, pt_i, nh_i, nt_i)
    rel_idx = (pr_i, nr_i)
    ent_slots = ((ea0, ea1, ea2, ea3), (eb0, eb1, eb2, eb3))
    rel_slots = ((la0, la1), (lb0, lb1))

    def fire(g, slot):
        r, c0 = g >> 2, (g & 3) * 16
        for j in range(4):
            iv = ent_idx[j][r, pl.ds(c0, 16)]
            rv = ((iv >> 12) << 11) + (iv & 2047)
            for i in range(16):
                pltpu.async_copy(
                    ent_h.at[rv[i]], ent_slots[slot][j].at[i], sem
                )
        for j in range(2):
            iv = rel_idx[j][r, pl.ds(c0, 16)]
            for i in range(16):
                pltpu.async_copy(
                    rel_h.at[iv[i]], rel_slots[slot][j].at[i], sem
                )

    def drain(slot):
        for j in range(4):
            pltpu.make_async_copy(
                ent_h.at[pl.ds(0, 16)], ent_slots[slot][j], sem
            ).wait()
        for j in range(2):
            pltpu.make_async_copy(
                rel_h.at[pl.ds(0, 16), pl.ds(0, 64)], rel_slots[slot][j], sem
            ).wait()

    def score16(g, slot):
        r, c0 = g >> 2, (g & 3) * 16
        ebufs = ent_slots[slot]
        lbufs = rel_slots[slot]
        hsel = [(ent_idx[j][r, pl.ds(c0, 16)] >> 11) & 1 for j in range(4)]
        total = 0.0
        for i in range(16):
            bits = [hsel[j][i] > 0 for j in range(4)]
            acc = jnp.zeros((16,), jnp.float32)
            for k in range(4):
                lo = pl.ds(k * 16, 16)
                hi = pl.ds(64 + k * 16, 16)
                hp = jnp.where(bits[0], ebufs[0][i, hi], ebufs[0][i, lo])
                tp = jnp.where(bits[1], ebufs[1][i, hi], ebufs[1][i, lo])
                hn = jnp.where(bits[2], ebufs[2][i, hi], ebufs[2][i, lo])
                tn = jnp.where(bits[3], ebufs[3][i, hi], ebufs[3][i, lo])
                rp = lbufs[0][i, lo]
                rn = lbufs[1][i, lo]
                acc = acc + (jnp.abs(hp + rp - tp) - jnp.abs(hn + rn - tn))
            total = total + jnp.maximum(jnp.sum(acc) + MARGIN, 0.0)
        return total

    fire(0, 0)

    def body(p, loss):
        g = p * 2
        fire(g + 1, 1)
        drain(0)
        loss = loss + score16(g, 0)
        pl.when(g + 2 < NG)(lambda: fire(g + 2, 0))
        drain(1)
        return loss + score16(g + 1, 1)

    loss = lax.fori_loop(0, NG // 2, body, 0.0)

    z16 = jnp.zeros((16,), jnp.float32)
    for r in range(8):
        for kk in range(4):
            ov[r, pl.ds(kk * 16, 16)] = z16
    ov[0, pl.ds(0, 16)] = jnp.full((16,), loss * 0.0625, jnp.float32)
    pltpu.sync_copy(ov, out_h.at[wid])


@functools.partial(jax.jit, static_argnums=())
def _run(ph, pt, pr, nh, nt, nr, ent, rel):
    mesh = plsc.VectorSubcoreMesh(core_axis_name="c", subcore_axis_name="s")
    k = pl.kernel(
        _tec_body,
        mesh=mesh,
        compiler_params=pltpu.CompilerParams(
            needs_layout_passes=False, use_tc_tiling_on_sc=True
        ),
        out_type=jax.ShapeDtypeStruct((NW, 8, 64), jnp.float32),
        scratch_types=(
            [pltpu.VMEM((8, 64), jnp.int32) for _ in range(6)]
            + [pltpu.VMEM((16, 128), jnp.float32) for _ in range(8)]
            + [pltpu.VMEM((16, 64), jnp.float32) for _ in range(4)]
            + [pltpu.VMEM((8, 64), jnp.float32), pltpu.SemaphoreType.DMA]
        ),
    )
    return k(ph, pt, pr, nh, nt, nr, ent, rel)


def _xpose_body(src_ref, dst_ref):
    x = src_ref[...]
    dst_ref[...] = jnp.concatenate(
        [x[:, : BLK // 2].T, x[:, BLK // 2 :].T], axis=1
    )


@jax.jit
def _tc_transpose(xt):
    v = xt.shape[1]
    nblk = pl.cdiv(v, BLK)
    return pl.pallas_call(
        _xpose_body,
        grid=(nblk,),
        in_specs=[pl.BlockSpec((D, BLK), lambda i: (0, i))],
        out_specs=pl.BlockSpec((BLK // 2, 128), lambda i: (i, 0)),
        out_shape=jax.ShapeDtypeStruct((nblk * (BLK // 2), 128), jnp.float32),
    )(xt)


def kernel(pos_h, pos_t, pos_r, neg_h, neg_t, neg_r, ent_embeddings, rel_embeddings):
    shp = (NW, 8, 64)
    ph = pos_h.reshape(shp).astype(jnp.int32)
    pt = pos_t.reshape(shp).astype(jnp.int32)
    pr = pos_r.reshape(shp).astype(jnp.int32)
    nh = neg_h.reshape(shp).astype(jnp.int32)
    nt = neg_t.reshape(shp).astype(jnp.int32)
    nr = neg_r.reshape(shp).astype(jnp.int32)
    ent_packed = _tc_transpose(ent_embeddings.T)
    out = _run(ph, pt, pr, nh, nt, nr, ent_packed, rel_embeddings)
    return jnp.sum(out)


# speedup vs baseline: 1.2174x; 1.1869x over previous
import functools

import jax
import jax.numpy as jnp
from jax import lax
from jax.experimental import pallas as pl
from jax.experimental.pallas import tpu as pltpu
from jax.experimental.pallas import tpu_sc as plsc

B = 16384
D = 64
MARGIN = 1.0
NC = 2
NS = 16
NW = NC * NS
BPW = B // NW
NG = BPW // 16
BLK = 8192
LOGB = BLK.bit_length() - 1
HMASK = BLK // 2 - 1


def _tec_body(ph_h, pt_h, pr_h, nh_h, nt_h, nr_h, ent_h, rel_h, out_h,
              ph_i, pt_i, pr_i, nh_i, nt_i, nr_i,
              ea0, ea1, ea2, ea3, eb0, eb1, eb2, eb3,
              la0, la1, lb0, lb1,
              ov, sem):
    wid = lax.axis_index("s") * NC + lax.axis_index("c")

    pltpu.sync_copy(ph_h.at[wid], ph_i)
    pltpu.sync_copy(pt_h.at[wid], pt_i)
    pltpu.sync_copy(pr_h.at[wid], pr_i)
    pltpu.sync_copy(nh_h.at[wid], nh_i)
    pltpu.sync_copy(nt_h.at[wid], nt_i)
    pltpu.sync_copy(nr_h.at[wid], nr_i)

    ent_idx = (ph_i, pt_i, nh_i, nt_i)
    rel_idx = (pr_i, nr_i)
    ent_slots = ((ea0, ea1, ea2, ea3), (eb0, eb1, eb2, eb3))
    rel_slots = ((la0, la1), (lb0, lb1))

    def fire(g, slot):
        r, c0 = g >> 2, (g & 3) * 16
        for j in range(4):
            iv = ent_idx[j][r, pl.ds(c0, 16)]
            rv = ((iv >> LOGB) << (LOGB - 1)) + (iv & HMASK)
            for i in range(16):
                pltpu.async_copy(
                    ent_h.at[rv[i]], ent_slots[slot][j].at[i], sem
                )
        for j in range(2):
            iv = rel_idx[j][r, pl.ds(c0, 16)]
            for i in range(16):
                pltpu.async_copy(
                    rel_h.at[iv[i]], rel_slots[slot][j].at[i], sem
                )

    def drain(slot):
        for j in range(4):
            pltpu.make_async_copy(
                ent_h.at[pl.ds(0, 16)], ent_slots[slot][j], sem
            ).wait()
        for j in range(2):
            pltpu.make_async_copy(
                rel_h.at[pl.ds(0, 16), pl.ds(0, 64)], rel_slots[slot][j], sem
            ).wait()

    def score16(g, slot):
        r, c0 = g >> 2, (g & 3) * 16
        ebufs = ent_slots[slot]
        lbufs = rel_slots[slot]
        hsel = [
            (ent_idx[j][r, pl.ds(c0, 16)] >> (LOGB - 1)) & 1 for j in range(4)
        ]
        total = 0.0
        for i in range(16):
            bits = [hsel[j][i] > 0 for j in range(4)]
            acc = jnp.zeros((16,), jnp.float32)
            for k in range(4):
                lo = pl.ds(k * 16, 16)
                hi = pl.ds(64 + k * 16, 16)
                hp = jnp.where(bits[0], ebufs[0][i, hi], ebufs[0][i, lo])
                tp = jnp.where(bits[1], ebufs[1][i, hi], ebufs[1][i, lo])
                hn = jnp.where(bits[2], ebufs[2][i, hi], ebufs[2][i, lo])
                tn = jnp.where(bits[3], ebufs[3][i, hi], ebufs[3][i, lo])
                rp = lbufs[0][i, lo]
                rn = lbufs[1][i, lo]
                acc = acc + (jnp.abs(hp + rp - tp) - jnp.abs(hn + rn - tn))
            total = total + jnp.maximum(jnp.sum(acc) + MARGIN, 0.0)
        return total

    fire(0, 0)

    def body(p, loss):
        g = p * 2
        fire(g + 1, 1)
        drain(0)
        loss = loss + score16(g, 0)
        pl.when(g + 2 < NG)(lambda: fire(g + 2, 0))
        drain(1)
        return loss + score16(g + 1, 1)

    loss = lax.fori_loop(0, NG // 2, body, 0.0)

    z16 = jnp.zeros((16,), jnp.float32)
    for r in range(8):
        for kk in range(4):
            ov[r, pl.ds(kk * 16, 16)] = z16
    ov[0, pl.ds(0, 16)] = jnp.full((16,), loss * 0.0625, jnp.float32)
    pltpu.sync_copy(ov, out_h.at[wid])


@functools.partial(jax.jit, static_argnums=())
def _run(ph, pt, pr, nh, nt, nr, ent, rel):
    mesh = plsc.VectorSubcoreMesh(core_axis_name="c", subcore_axis_name="s")
    k = pl.kernel(
        _tec_body,
        mesh=mesh,
        compiler_params=pltpu.CompilerParams(
            needs_layout_passes=False, use_tc_tiling_on_sc=True
        ),
        out_type=jax.ShapeDtypeStruct((NW, 8, 64), jnp.float32),
        scratch_types=(
            [pltpu.VMEM((8, 64), jnp.int32) for _ in range(6)]
            + [pltpu.VMEM((16, 128), jnp.float32) for _ in range(8)]
            + [pltpu.VMEM((16, 64), jnp.float32) for _ in range(4)]
            + [pltpu.VMEM((8, 64), jnp.float32), pltpu.SemaphoreType.DMA]
        ),
    )
    return k(ph, pt, pr, nh, nt, nr, ent, rel)


def _xpose_body(src_ref, dst_ref):
    x = src_ref[...]
    dst_ref[...] = jnp.concatenate(
        [x[:, : BLK // 2].T, x[:, BLK // 2 :].T], axis=1
    )


@jax.jit
def _tc_transpose(xt):
    v = xt.shape[1]
    nblk = pl.cdiv(v, BLK)
    return pl.pallas_call(
        _xpose_body,
        grid=(nblk,),
        in_specs=[pl.BlockSpec((D, BLK), lambda i: (0, i))],
        out_specs=pl.BlockSpec((BLK // 2, 128), lambda i: (i, 0)),
        out_shape=jax.ShapeDtypeStruct((nblk * (BLK // 2), 128), jnp.float32),
    )(xt)


def kernel(pos_h, pos_t, pos_r, neg_h, neg_t, neg_r, ent_embeddings, rel_embeddings):
    shp = (NW, 8, 64)
    ph = pos_h.reshape(shp).astype(jnp.int32)
    pt = pos_t.reshape(shp).astype(jnp.int32)
    pr = pos_r.reshape(shp).astype(jnp.int32)
    nh = neg_h.reshape(shp).astype(jnp.int32)
    nt = neg_t.reshape(shp).astype(jnp.int32)
    nr = neg_r.reshape(shp).astype(jnp.int32)
    ent_packed = _tc_transpose(ent_embeddings.T)
    out = _run(ph, pt, pr, nh, nt, nr, ent_packed, rel_embeddings)
    return jnp.sum(out)


# speedup vs baseline: 1.3416x; 1.1020x over previous
import functools

import jax
import jax.numpy as jnp
from jax import lax
from jax.experimental import pallas as pl
from jax.experimental.pallas import tpu as pltpu
from jax.experimental.pallas import tpu_sc as plsc

B = 16384
D = 64
MARGIN = 1.0
NC = 2
NS = 16
NW = NC * NS
BPW = B // NW
NG = BPW // 16
BLK = 16384
LOGB = BLK.bit_length() - 1
HMASK = BLK // 2 - 1


def _tec_body(ph_h, pt_h, pr_h, nh_h, nt_h, nr_h, ent_h, rel_h, out_h,
              ph_i, pt_i, pr_i, nh_i, nt_i, nr_i,
              ea0, ea1, ea2, ea3, eb0, eb1, eb2, eb3,
              la0, la1, lb0, lb1,
              ov, sem):
    wid = lax.axis_index("s") * NC + lax.axis_index("c")

    pltpu.sync_copy(ph_h.at[wid], ph_i)
    pltpu.sync_copy(pt_h.at[wid], pt_i)
    pltpu.sync_copy(pr_h.at[wid], pr_i)
    pltpu.sync_copy(nh_h.at[wid], nh_i)
    pltpu.sync_copy(nt_h.at[wid], nt_i)
    pltpu.sync_copy(nr_h.at[wid], nr_i)

    ent_idx = (ph_i, pt_i, nh_i, nt_i)
    rel_idx = (pr_i, nr_i)
    ent_slots = ((ea0, ea1, ea2, ea3), (eb0, eb1, eb2, eb3))
    rel_slots = ((la0, la1), (lb0, lb1))

    def fire(g, slot):
        r, c0 = g >> 2, (g & 3) * 16
        for j in range(4):
            iv = ent_idx[j][r, pl.ds(c0, 16)]
            rv = ((iv >> LOGB) << (LOGB - 1)) + (iv & HMASK)
            for i in range(16):
                pltpu.async_copy(
                    ent_h.at[rv[i]], ent_slots[slot][j].at[i], sem
                )
        for j in range(2):
            iv = rel_idx[j][r, pl.ds(c0, 16)]
            for i in range(16):
                pltpu.async_copy(
                    rel_h.at[iv[i]], rel_slots[slot][j].at[i], sem
                )

    def drain(slot):
        for j in range(4):
            pltpu.make_async_copy(
                ent_h.at[pl.ds(0, 16)], ent_slots[slot][j], sem
            ).wait()
        for j in range(2):
            pltpu.make_async_copy(
                rel_h.at[pl.ds(0, 16), pl.ds(0, 64)], rel_slots[slot][j], sem
            ).wait()

    def score16(g, slot):
        r, c0 = g >> 2, (g & 3) * 16
        ebufs = ent_slots[slot]
        lbufs = rel_slots[slot]
        hsel = [
            (ent_idx[j][r, pl.ds(c0, 16)] >> (LOGB - 1)) & 1 for j in range(4)
        ]
        total = 0.0
        for i in range(16):
            bits = [hsel[j][i] > 0 for j in range(4)]
            acc = jnp.zeros((16,), jnp.float32)
            for k in range(4):
                lo = pl.ds(k * 16, 16)
                hi = pl.ds(64 + k * 16, 16)
                hp = jnp.where(bits[0], ebufs[0][i, hi], ebufs[0][i, lo])
                tp = jnp.where(bits[1], ebufs[1][i, hi], ebufs[1][i, lo])
                hn = jnp.where(bits[2], ebufs[2][i, hi], ebufs[2][i, lo])
                tn = jnp.where(bits[3], ebufs[3][i, hi], ebufs[3][i, lo])
                rp = lbufs[0][i, lo]
                rn = lbufs[1][i, lo]
                acc = acc + (jnp.abs(hp + rp - tp) - jnp.abs(hn + rn - tn))
            total = total + jnp.maximum(jnp.sum(acc) + MARGIN, 0.0)
        return total

    fire(0, 0)

    def body(p, loss):
        g = p * 2
        fire(g + 1, 1)
        drain(0)
        loss = loss + score16(g, 0)
        pl.when(g + 2 < NG)(lambda: fire(g + 2, 0))
        drain(1)
        return loss + score16(g + 1, 1)

    loss = lax.fori_loop(0, NG // 2, body, 0.0)

    z16 = jnp.zeros((16,), jnp.float32)
    for r in range(8):
        for kk in range(4):
            ov[r, pl.ds(kk * 16, 16)] = z16
    ov[0, pl.ds(0, 16)] = jnp.full((16,), loss * 0.0625, jnp.float32)
    pltpu.sync_copy(ov, out_h.at[wid])


@functools.partial(jax.jit, static_argnums=())
def _run(ph, pt, pr, nh, nt, nr, ent, rel):
    mesh = plsc.VectorSubcoreMesh(core_axis_name="c", subcore_axis_name="s")
    k = pl.kernel(
        _tec_body,
        mesh=mesh,
        compiler_params=pltpu.CompilerParams(
            needs_layout_passes=False, use_tc_tiling_on_sc=True
        ),
        out_type=jax.ShapeDtypeStruct((NW, 8, 64), jnp.float32),
        scratch_types=(
            [pltpu.VMEM((8, 64), jnp.int32) for _ in range(6)]
            + [pltpu.VMEM((16, 128), jnp.float32) for _ in range(8)]
            + [pltpu.VMEM((16, 64), jnp.float32) for _ in range(4)]
            + [pltpu.VMEM((8, 64), jnp.float32), pltpu.SemaphoreType.DMA]
        ),
    )
    return k(ph, pt, pr, nh, nt, nr, ent, rel)


def _xpose_body(src_ref, dst_ref):
    x = src_ref[...]
    dst_ref[...] = jnp.concatenate(
        [x[:, : BLK // 2].T, x[:, BLK // 2 :].T], axis=1
    )


@jax.jit
def _tc_transpose(xt):
    v = xt.shape[1]
    nblk = pl.cdiv(v, BLK)
    return pl.pallas_call(
        _xpose_body,
        grid=(nblk,),
        in_specs=[pl.BlockSpec((D, BLK), lambda i: (0, i))],
        out_specs=pl.BlockSpec((BLK // 2, 128), lambda i: (i, 0)),
        out_shape=jax.ShapeDtypeStruct((nblk * (BLK // 2), 128), jnp.float32),
    )(xt)


def kernel(pos_h, pos_t, pos_r, neg_h, neg_t, neg_r, ent_embeddings, rel_embeddings):
    shp = (NW, 8, 64)
    ph = pos_h.reshape(shp).astype(jnp.int32)
    pt = pos_t.reshape(shp).astype(jnp.int32)
    pr = pos_r.reshape(shp).astype(jnp.int32)
    nh = neg_h.reshape(shp).astype(jnp.int32)
    nt = neg_t.reshape(shp).astype(jnp.int32)
    nr = neg_r.reshape(shp).astype(jnp.int32)
    ent_packed = _tc_transpose(ent_embeddings.T)
    out = _run(ph, pt, pr, nh, nt, nr, ent_packed, rel_embeddings)
    return jnp.sum(out)


# speedup vs baseline: 1.4098x; 1.0509x over previous
import functools

import jax
import jax.numpy as jnp
from jax import lax
from jax.experimental import pallas as pl
from jax.experimental.pallas import tpu as pltpu
from jax.experimental.pallas import tpu_sc as plsc

B = 16384
D = 64
MARGIN = 1.0
NC = 2
NS = 16
NW = NC * NS
BPW = B // NW
NG = BPW // 16
BLK = 32768
LOGB = BLK.bit_length() - 1
HMASK = BLK // 2 - 1


def _tec_body(ph_h, pt_h, pr_h, nh_h, nt_h, nr_h, ent_h, rel_h, out_h,
              ph_i, pt_i, pr_i, nh_i, nt_i, nr_i,
              ea0, ea1, ea2, ea3, eb0, eb1, eb2, eb3,
              la0, la1, lb0, lb1,
              ov, sem):
    wid = lax.axis_index("s") * NC + lax.axis_index("c")

    pltpu.sync_copy(ph_h.at[wid], ph_i)
    pltpu.sync_copy(pt_h.at[wid], pt_i)
    pltpu.sync_copy(pr_h.at[wid], pr_i)
    pltpu.sync_copy(nh_h.at[wid], nh_i)
    pltpu.sync_copy(nt_h.at[wid], nt_i)
    pltpu.sync_copy(nr_h.at[wid], nr_i)

    ent_idx = (ph_i, pt_i, nh_i, nt_i)
    rel_idx = (pr_i, nr_i)
    ent_slots = ((ea0, ea1, ea2, ea3), (eb0, eb1, eb2, eb3))
    rel_slots = ((la0, la1), (lb0, lb1))

    def fire(g, slot):
        r, c0 = g >> 2, (g & 3) * 16
        for j in range(4):
            iv = ent_idx[j][r, pl.ds(c0, 16)]
            rv = ((iv >> LOGB) << (LOGB - 1)) + (iv & HMASK)
            for i in range(16):
                pltpu.async_copy(
                    ent_h.at[rv[i]], ent_slots[slot][j].at[i], sem
                )
        for j in range(2):
            iv = rel_idx[j][r, pl.ds(c0, 16)]
            for i in range(16):
                pltpu.async_copy(
                    rel_h.at[iv[i]], rel_slots[slot][j].at[i], sem
                )

    def drain(slot):
        for j in range(4):
            pltpu.make_async_copy(
                ent_h.at[pl.ds(0, 16)], ent_slots[slot][j], sem
            ).wait()
        for j in range(2):
            pltpu.make_async_copy(
                rel_h.at[pl.ds(0, 16), pl.ds(0, 64)], rel_slots[slot][j], sem
            ).wait()

    def score16(g, slot):
        r, c0 = g >> 2, (g & 3) * 16
        ebufs = ent_slots[slot]
        lbufs = rel_slots[slot]
        hsel = [
            (ent_idx[j][r, pl.ds(c0, 16)] >> (LOGB - 1)) & 1 for j in range(4)
        ]
        total = 0.0
        for i in range(16):
            bits = [hsel[j][i] > 0 for j in range(4)]
            acc = jnp.zeros((16,), jnp.float32)
            for k in range(4):
                lo = pl.ds(k * 16, 16)
                hi = pl.ds(64 + k * 16, 16)
                hp = jnp.where(bits[0], ebufs[0][i, hi], ebufs[0][i, lo])
                tp = jnp.where(bits[1], ebufs[1][i, hi], ebufs[1][i, lo])
                hn = jnp.where(bits[2], ebufs[2][i, hi], ebufs[2][i, lo])
                tn = jnp.where(bits[3], ebufs[3][i, hi], ebufs[3][i, lo])
                rp = lbufs[0][i, lo]
                rn = lbufs[1][i, lo]
                acc = acc + (jnp.abs(hp + rp - tp) - jnp.abs(hn + rn - tn))
            total = total + jnp.maximum(jnp.sum(acc) + MARGIN, 0.0)
        return total

    fire(0, 0)

    def body(p, loss):
        g = p * 2
        fire(g + 1, 1)
        drain(0)
        loss = loss + score16(g, 0)
        pl.when(g + 2 < NG)(lambda: fire(g + 2, 0))
        drain(1)
        return loss + score16(g + 1, 1)

    loss = lax.fori_loop(0, NG // 2, body, 0.0)

    z16 = jnp.zeros((16,), jnp.float32)
    for r in range(8):
        for kk in range(4):
            ov[r, pl.ds(kk * 16, 16)] = z16
    ov[0, pl.ds(0, 16)] = jnp.full((16,), loss * 0.0625, jnp.float32)
    pltpu.sync_copy(ov, out_h.at[wid])


@functools.partial(jax.jit, static_argnums=())
def _run(ph, pt, pr, nh, nt, nr, ent, rel):
    mesh = plsc.VectorSubcoreMesh(core_axis_name="c", subcore_axis_name="s")
    k = pl.kernel(
        _tec_body,
        mesh=mesh,
        compiler_params=pltpu.CompilerParams(
            needs_layout_passes=False, use_tc_tiling_on_sc=True
        ),
        out_type=jax.ShapeDtypeStruct((NW, 8, 64), jnp.float32),
        scratch_types=(
            [pltpu.VMEM((8, 64), jnp.int32) for _ in range(6)]
            + [pltpu.VMEM((16, 128), jnp.float32) for _ in range(8)]
            + [pltpu.VMEM((16, 64), jnp.float32) for _ in range(4)]
            + [pltpu.VMEM((8, 64), jnp.float32), pltpu.SemaphoreType.DMA]
        ),
    )
    return k(ph, pt, pr, nh, nt, nr, ent, rel)


def _xpose_body(src_ref, dst_ref):
    x = src_ref[...]
    dst_ref[...] = jnp.concatenate(
        [x[:, : BLK // 2].T, x[:, BLK // 2 :].T], axis=1
    )


@jax.jit
def _tc_transpose(xt):
    v = xt.shape[1]
    nblk = pl.cdiv(v, BLK)
    return pl.pallas_call(
        _xpose_body,
        grid=(nblk,),
        in_specs=[pl.BlockSpec((D, BLK), lambda i: (0, i))],
        out_specs=pl.BlockSpec((BLK // 2, 128), lambda i: (i, 0)),
        out_shape=jax.ShapeDtypeStruct((nblk * (BLK // 2), 128), jnp.float32),
    )(xt)


def kernel(pos_h, pos_t, pos_r, neg_h, neg_t, neg_r, ent_embeddings, rel_embeddings):
    shp = (NW, 8, 64)
    ph = pos_h.reshape(shp).astype(jnp.int32)
    pt = pos_t.reshape(shp).astype(jnp.int32)
    pr = pos_r.reshape(shp).astype(jnp.int32)
    nh = neg_h.reshape(shp).astype(jnp.int32)
    nt = neg_t.reshape(shp).astype(jnp.int32)
    nr = neg_r.reshape(shp).astype(jnp.int32)
    ent_packed = _tc_transpose(ent_embeddings.T)
    out = _run(ph, pt, pr, nh, nt, nr, ent_packed, rel_embeddings)
    return jnp.sum(out)
